# baseline (speedup 1.0000x reference)
import functools

import jax
import jax.numpy as jnp
import numpy as np
from jax import lax
from jax.experimental import pallas as pl
from jax.experimental.pallas import tpu as pltpu
from jax.experimental.pallas import tpu_sc as plsc

B = 4096
L = 200
D = 64
NUM_CORES = 2
NUM_SUBCORES = 16
NW = NUM_CORES * NUM_SUBCORES
NB = B // NW


def _pe_table() -> jnp.ndarray:
    pos = np.arange(L, dtype=np.float32)[:, None]
    i = np.arange(0, D, 2, dtype=np.float32)
    div = np.exp(-np.log(10000.0) * i / float(D))
    pe = np.zeros((L, D), dtype=np.float32)
    pe[:, 0::2] = np.sin(pos * div)
    pe[:, 1::2] = np.cos(pos * div)
    return jnp.asarray(pe)


_mesh = plsc.VectorSubcoreMesh(
    core_axis_name="c", subcore_axis_name="s",
    num_cores=NUM_CORES, num_subcores=NUM_SUBCORES)


@functools.partial(
    pl.kernel,
    out_type=jax.ShapeDtypeStruct((L, D, B), jnp.float32),
    mesh=_mesh,
    scratch_types=[
        pltpu.VMEM((2, NB), jnp.int32),
        pltpu.VMEM((2, NB, D), jnp.float32),
        pltpu.VMEM((2, D, NB), jnp.float32),
        pltpu.VMEM((L, D), jnp.float32),
        [pltpu.SemaphoreType.DMA] * 2,
        [pltpu.SemaphoreType.DMA] * 2,
        [pltpu.SemaphoreType.DMA] * 2,
    ],
    compiler_params=pltpu.CompilerParams(
        use_tc_tiling_on_sc=False, needs_layout_passes=False),
)
def _emb_sc(tgt_t_hbm, pe_hbm, table_hbm, out_hbm,
            idx_v, rows_v, tbuf_v, pe_v, isem, gsem, osem):
    wid = lax.axis_index("s") * NUM_CORES + lax.axis_index("c")
    b0 = wid * NB
    pltpu.sync_copy(pe_hbm, pe_v)

    def idx_copy(l, b):
        return pltpu.make_async_copy(
            tgt_t_hbm.at[l, pl.ds(b0, NB)], idx_v.at[b], isem[b])

    def gather(b):
        return pltpu.make_async_copy(
            table_hbm.at[idx_v.at[b]], rows_v.at[b], gsem[b])

    def out_copy(l, b):
        return pltpu.make_async_copy(
            tbuf_v.at[b], out_hbm.at[l, :, pl.ds(b0, NB)], osem[b])

    idx_copy(0, 0).start()
    idx_copy(1, 1).start()
    idx_copy(0, 0).wait()
    gather(0).start()

    c_iota = [lax.iota(jnp.int32, 16) + 16 * q for q in range(D // 16)]

    def step(l, b):
        @pl.when(l + 1 < L)
        def _():
            idx_copy(l + 1, 1 - b).wait()
            gather(1 - b).start()

        gather(b).wait()

        @pl.when(l + 2 < L)
        def _():
            idx_copy(l + 2, b).start()

        @pl.when(l >= 2)
        def _():
            out_copy(l - 2, b).wait()

        pe_r = [pe_v[l, pl.ds(16 * q, 16)] for q in range(D // 16)]

        def fuse(j, _):
            col = jnp.full((16,), 0, jnp.int32) + j
            for q in range(D // 16):
                v = rows_v[b, j, pl.ds(16 * q, 16)] * 8.0 + pe_r[q]
                plsc.store_scatter(tbuf_v.at[b], [c_iota[q], col], v)
            return 0

        lax.fori_loop(0, NB, fuse, 0, unroll=4)
        out_copy(l, b).start()

    def outer(g, _):
        for b in range(2):
            step(g * 2 + b, b)
        return 0

    lax.fori_loop(0, L // 2, outer, 0)

    out_copy(L - 2, 0).wait()
    out_copy(L - 1, 1).wait()


def _mask_body(o_ref):
    r = lax.broadcasted_iota(jnp.int32, (L, L), 0)
    c = lax.broadcasted_iota(jnp.int32, (L, L), 1)
    o_ref[...] = jnp.where(r >= c, jnp.float32(0.0), jnp.float32(-jnp.inf))


_mask_call = pl.pallas_call(
    _mask_body,
    out_shape=jax.ShapeDtypeStruct((L, L), jnp.float32),
)


def kernel(tgt, table):
    tgt_t = tgt.astype(jnp.int32).T
    out_t = _emb_sc(tgt_t, _pe_table(), table)
    emb = jnp.transpose(out_t, (2, 0, 1))
    return emb, _mask_call()


# speedup vs baseline: 1.6362x; 1.6362x over previous
import functools

import jax
import jax.numpy as jnp
import numpy as np
from jax import lax
from jax.experimental import pallas as pl
from jax.experimental.pallas import tpu as pltpu
from jax.experimental.pallas import tpu_sc as plsc

B = 4096
L = 200
D = 64
NUM_CORES = 2
NUM_SUBCORES = 16
NW = NUM_CORES * NUM_SUBCORES
NB = B // NW


def _pe_table() -> jnp.ndarray:
    pos = np.arange(L, dtype=np.float32)[:, None]
    i = np.arange(0, D, 2, dtype=np.float32)
    div = np.exp(-np.log(10000.0) * i / float(D))
    pe = np.zeros((L, D), dtype=np.float32)
    pe[:, 0::2] = np.sin(pos * div)
    pe[:, 1::2] = np.cos(pos * div)
    return jnp.asarray(pe)


_mesh = plsc.VectorSubcoreMesh(
    core_axis_name="c", subcore_axis_name="s",
    num_cores=NUM_CORES, num_subcores=NUM_SUBCORES)


@functools.partial(
    pl.kernel,
    out_type=jax.ShapeDtypeStruct((L, B, D), jnp.float32),
    mesh=_mesh,
    scratch_types=[
        pltpu.VMEM((2, NB), jnp.int32),
        pltpu.VMEM((2, NB, D), jnp.float32),
        pltpu.VMEM((L, D), jnp.float32),
        [pltpu.SemaphoreType.DMA] * 2,
        [pltpu.SemaphoreType.DMA] * 2,
        [pltpu.SemaphoreType.DMA] * 2,
    ],
    compiler_params=pltpu.CompilerParams(
        use_tc_tiling_on_sc=False, needs_layout_passes=False),
)
def _emb_sc(tgt_t_hbm, pe_hbm, table_hbm, out_hbm,
            idx_v, rows_v, pe_v, isem, gsem, osem):
    wid = lax.axis_index("s") * NUM_CORES + lax.axis_index("c")
    b0 = wid * NB
    pltpu.sync_copy(pe_hbm, pe_v)

    def idx_copy(l, b):
        return pltpu.make_async_copy(
            tgt_t_hbm.at[l, pl.ds(b0, NB)], idx_v.at[b], isem[b])

    def gather(b):
        return pltpu.make_async_copy(
            table_hbm.at[idx_v.at[b]], rows_v.at[b], gsem[b])

    def out_copy(l, b):
        return pltpu.make_async_copy(
            rows_v.at[b], out_hbm.at[l, pl.ds(b0, NB)], osem[b])

    idx_copy(0, 0).start()
    idx_copy(1, 1).start()
    idx_copy(0, 0).wait()
    gather(0).start()

    def step(l, b):
        @pl.when(l + 1 < L)
        def _():
            idx_copy(l + 1, 1 - b).wait()
            gather(1 - b).start()

        gather(b).wait()

        @pl.when(l + 2 < L)
        def _():
            idx_copy(l + 2, b).start()

        @pl.when(l >= 2)
        def _():
            out_copy(l - 2, b).wait()

        pe_r = [pe_v[l, pl.ds(16 * q, 16)] for q in range(D // 16)]

        def fuse(j, _):
            for q in range(D // 16):
                sl = pl.ds(16 * q, 16)
                rows_v[b, j, sl] = rows_v[b, j, sl] * 8.0 + pe_r[q]
            return 0

        lax.fori_loop(0, NB, fuse, 0, unroll=4)
        out_copy(l, b).start()

    def outer(g, _):
        for b in range(2):
            step(g * 2 + b, b)
        return 0

    lax.fori_loop(0, L // 2, outer, 0)

    out_copy(L - 2, 0).wait()
    out_copy(L - 1, 1).wait()


def _mask_body(o_ref):
    r = lax.broadcasted_iota(jnp.int32, (L, L), 0)
    c = lax.broadcasted_iota(jnp.int32, (L, L), 1)
    o_ref[...] = jnp.where(r >= c, jnp.float32(0.0), jnp.float32(-jnp.inf))


_mask_call = pl.pallas_call(
    _mask_body,
    out_shape=jax.ShapeDtypeStruct((L, L), jnp.float32),
)


def kernel(tgt, table):
    tgt_t = tgt.astype(jnp.int32).T
    out_t = _emb_sc(tgt_t, _pe_table(), table)
    emb = jnp.transpose(out_t, (1, 0, 2))
    return emb, _mask_call()
